# Initial kernel scaffold; baseline (speedup 1.0000x reference)
import jax
import jax.numpy as jnp
from jax.experimental import pallas as pl


def kernel(x, mask_token):
    raise NotImplementedError("write your pallas kernel here")


# speedup vs baseline: 1.0977x; 1.0977x over previous
import numpy as np
import jax
import jax.numpy as jnp
from jax.experimental import pallas as pl
from jax.experimental.pallas import tpu as pltpu

_MASK_RATIO = 0.5
_B, _T, _P, _D = 8, 16, 576, 768
_N = _B * _T

_MASK = np.asarray(jax.random.uniform(jax.random.key(42), (_B, _T)) < _MASK_RATIO)
_MASKED = np.nonzero(_MASK.ravel())[0].astype(np.int32)
_UNMASKED = np.nonzero(~_MASK.ravel())[0].astype(np.int32)
_M = int(_MASKED.size)

_PIN = int(_UNMASKED[0]) if _UNMASKED.size else 0
_SRC = np.concatenate([np.full(_M, _PIN, np.int32), _UNMASKED]).astype(np.int32)
_DST = np.concatenate([_MASKED, _UNMASKED]).astype(np.int32)


def _body(src_ref, dst_ref, x_ref, tok_ref, out_ref):
    i = pl.program_id(0)
    tok = tok_ref[0, :]
    out_ref[0] = jnp.where(i < _M, tok[None, :], x_ref[0])


def kernel(x, mask_token):
    x3 = x.reshape(_N, _P, _D)
    tok = mask_token.reshape(1, _D)
    grid_spec = pltpu.PrefetchScalarGridSpec(
        num_scalar_prefetch=2,
        grid=(_N,),
        in_specs=[
            pl.BlockSpec((1, _P, _D), lambda i, src, dst: (src[i], 0, 0)),
            pl.BlockSpec((1, _D), lambda i, src, dst: (0, 0)),
        ],
        out_specs=pl.BlockSpec((1, _P, _D), lambda i, src, dst: (dst[i], 0, 0)),
    )
    out3 = pl.pallas_call(
        _body,
        grid_spec=grid_spec,
        out_shape=jax.ShapeDtypeStruct((_N, _P, _D), x.dtype),
    )(jnp.asarray(_SRC), jnp.asarray(_DST), x3, tok)
    return out3.reshape(_B, _T, _P, _D)
